# baseline (speedup 1.0000x reference)
import functools

import jax
import jax.numpy as jnp
from jax import lax
from jax.experimental import pallas as pl
from jax.experimental.pallas import tpu as pltpu
from jax.experimental.pallas import tpu_sc as plsc

_R = 128
_C = 32768
_L = 16
_NW = 32
_RPW = _R // _NW
_NCHUNK = _C // _L


def _row_argmax(buf):
    def step(i, carry):
        m, c = carry
        x = buf[pl.ds(i * _L, _L)]
        gt = x > m
        m = jnp.where(gt, x, m)
        c = jnp.where(gt, i, c)
        return m, c

    m0 = buf[pl.ds(0, _L)]
    c0 = jnp.zeros((_L,), jnp.int32)
    m, c = lax.fori_loop(1, _NCHUNK, step, (m0, c0), unroll=8)
    bm = m[0]
    bi = c[0] * _L
    for j in range(1, _L):
        mv = m[j]
        iv = c[j] * _L + j
        better = (mv > bm) | ((mv == bm) & (iv < bi))
        bm = jnp.where(better, mv, bm)
        bi = jnp.where(better, iv, bi)
    return bi


@functools.partial(
    pl.kernel,
    mesh=plsc.VectorSubcoreMesh(core_axis_name="c", subcore_axis_name="s"),
    out_type=jax.ShapeDtypeStruct((_NW * 8,), jnp.int32),
    scratch_types=[
        pltpu.VMEM((_C,), jnp.float32),
        pltpu.VMEM((_C,), jnp.float32),
        pltpu.VMEM((_L,), jnp.int32),
        pltpu.SemaphoreType.DMA,
        pltpu.SemaphoreType.DMA,
    ],
)
def _argmax_sc(x_hbm, out_hbm, buf0, buf1, res, sem0, sem1):
    cid = lax.axis_index("c")
    sid = lax.axis_index("s")
    wid = cid * 16 + sid
    row0 = wid * _RPW
    bufs = (buf0, buf1)
    sems = (sem0, sem1)
    lanes = lax.iota(jnp.int32, _L)

    copies = [None, None]
    copies[0] = pltpu.async_copy(x_hbm.at[row0], buf0, sem0)
    resvec = jnp.zeros((_L,), jnp.int32)
    for r in range(_RPW):
        cur = r % 2
        nxt = (r + 1) % 2
        if r + 1 < _RPW:
            copies[nxt] = pltpu.async_copy(
                x_hbm.at[row0 + r + 1], bufs[nxt], sems[nxt])
        copies[cur].wait()
        a = _row_argmax(bufs[cur])
        resvec = jnp.where(lanes == r, a, resvec)
    res[...] = resvec
    pltpu.sync_copy(res.at[pl.ds(0, 8)], out_hbm.at[pl.ds(wid * 8, 8)])


def kernel(tensor):
    out = _argmax_sc(tensor)
    return out.reshape(_NW, 8)[:, :_RPW].reshape(_R)


# speedup vs baseline: 1.0984x; 1.0984x over previous
import functools

import jax
import jax.numpy as jnp
from jax import lax
from jax.experimental import pallas as pl
from jax.experimental.pallas import tpu as pltpu
from jax.experimental.pallas import tpu_sc as plsc

_R = 128
_C = 32768
_L = 16
_NW = 32
_RPW = _R // _NW
_NCHUNK = _C // _L


_NACC = 4
_NITER = _NCHUNK // _NACC


def _row_argmax(buf):
    lanes = lax.iota(jnp.int32, _L)

    def step(t, carry):
        ms, cs = carry
        tvec = lax.broadcast(t, (_L,))
        base = t * (_L * _NACC)
        nms, ncs = [], []
        for k in range(_NACC):
            x = buf[pl.ds(base + k * _L, _L)]
            gt = x > ms[k]
            nms.append(jnp.where(gt, x, ms[k]))
            ncs.append(jnp.where(gt, tvec, cs[k]))
        return tuple(nms), tuple(ncs)

    ms0 = tuple(buf[pl.ds(k * _L, _L)] for k in range(_NACC))
    cs0 = tuple(jnp.zeros((_L,), jnp.int32) for _ in range(_NACC))
    ms, cs = lax.fori_loop(1, _NITER, step, (ms0, cs0), unroll=4)

    bm = ms[0]
    bidx = cs[0] * (_NACC * _L) + lanes
    for k in range(1, _NACC):
        idxk = cs[k] * (_NACC * _L) + (k * _L) + lanes
        better = (ms[k] > bm) | ((ms[k] == bm) & (idxk < bidx))
        bm = jnp.where(better, ms[k], bm)
        bidx = jnp.where(better, idxk, bidx)
    sm = bm[0]
    si = bidx[0]
    for j in range(1, _L):
        mv = bm[j]
        iv = bidx[j]
        better = (mv > sm) | ((mv == sm) & (iv < si))
        sm = jnp.where(better, mv, sm)
        si = jnp.where(better, iv, si)
    return si


@functools.partial(
    pl.kernel,
    mesh=plsc.VectorSubcoreMesh(core_axis_name="c", subcore_axis_name="s"),
    out_type=jax.ShapeDtypeStruct((_NW * 8,), jnp.int32),
    scratch_types=[
        pltpu.VMEM((_C,), jnp.float32),
        pltpu.VMEM((_C,), jnp.float32),
        pltpu.VMEM((_L,), jnp.int32),
        pltpu.SemaphoreType.DMA,
        pltpu.SemaphoreType.DMA,
    ],
)
def _argmax_sc(x_hbm, out_hbm, buf0, buf1, res, sem0, sem1):
    cid = lax.axis_index("c")
    sid = lax.axis_index("s")
    wid = cid * 16 + sid
    row0 = wid * _RPW
    bufs = (buf0, buf1)
    sems = (sem0, sem1)
    lanes = lax.iota(jnp.int32, _L)

    copies = [None, None]
    copies[0] = pltpu.async_copy(x_hbm.at[row0], buf0, sem0)
    resvec = jnp.zeros((_L,), jnp.int32)
    for r in range(_RPW):
        cur = r % 2
        nxt = (r + 1) % 2
        if r + 1 < _RPW:
            copies[nxt] = pltpu.async_copy(
                x_hbm.at[row0 + r + 1], bufs[nxt], sems[nxt])
        copies[cur].wait()
        a = _row_argmax(bufs[cur])
        resvec = jnp.where(lanes == r, a, resvec)
    res[...] = resvec
    pltpu.sync_copy(res.at[pl.ds(0, 8)], out_hbm.at[pl.ds(wid * 8, 8)])


def kernel(tensor):
    out = _argmax_sc(tensor)
    return out.reshape(_NW, 8)[:, :_RPW].reshape(_R)


# speedup vs baseline: 1.1442x; 1.0416x over previous
import functools

import jax
import jax.numpy as jnp
from jax import lax
from jax.experimental import pallas as pl
from jax.experimental.pallas import tpu as pltpu
from jax.experimental.pallas import tpu_sc as plsc

_R = 128
_C = 32768
_L = 16
_NW = 32
_RPW = _R // _NW
_NCHUNK = _C // _L


_NACC = 4
_NITER = _NCHUNK // _NACC


def _row_argmax(buf):
    lanes = lax.iota(jnp.int32, _L)

    def step(t, carry):
        ms, cs = carry
        tvec = lax.broadcast(t, (_L,))
        base = t * (_L * _NACC)
        nms, ncs = [], []
        for k in range(_NACC):
            x = buf[pl.ds(base + k * _L, _L)]
            gt = x > ms[k]
            nms.append(jnp.where(gt, x, ms[k]))
            ncs.append(jnp.where(gt, tvec, cs[k]))
        return tuple(nms), tuple(ncs)

    ms0 = tuple(buf[pl.ds(k * _L, _L)] for k in range(_NACC))
    cs0 = tuple(jnp.zeros((_L,), jnp.int32) for _ in range(_NACC))
    ms, cs = lax.fori_loop(1, _NITER, step, (ms0, cs0), unroll=4)

    bm = ms[0]
    bidx = cs[0] * (_NACC * _L) + lanes
    for k in range(1, _NACC):
        idxk = cs[k] * (_NACC * _L) + (k * _L) + lanes
        better = (ms[k] > bm) | ((ms[k] == bm) & (idxk < bidx))
        bm = jnp.where(better, ms[k], bm)
        bidx = jnp.where(better, idxk, bidx)
    sm = bm[0]
    si = bidx[0]
    for j in range(1, _L):
        mv = bm[j]
        iv = bidx[j]
        better = (mv > sm) | ((mv == sm) & (iv < si))
        sm = jnp.where(better, mv, sm)
        si = jnp.where(better, iv, si)
    return si


@functools.partial(
    pl.kernel,
    mesh=plsc.VectorSubcoreMesh(core_axis_name="c", subcore_axis_name="s"),
    out_type=jax.ShapeDtypeStruct((_R,), jnp.int32),
    scratch_types=[
        pltpu.VMEM((_C,), jnp.float32),
        pltpu.VMEM((_C,), jnp.float32),
        pltpu.VMEM((_L,), jnp.int32),
        pltpu.VMEM((4 * _L,), jnp.int32),
        pltpu.VMEM((_L,), jnp.int32),
        pltpu.VMEM_SHARED((16 * _L,), jnp.int32),
        pltpu.SemaphoreType.DMA,
        pltpu.SemaphoreType.DMA,
    ],
)
def _argmax_sc(x_hbm, out_hbm, buf0, buf1, res, gbuf, outv, shared,
               sem0, sem1):
    cid = lax.axis_index("c")
    sid = lax.axis_index("s")
    wid = cid * 16 + sid
    row0 = wid * _RPW
    bufs = (buf0, buf1)
    sems = (sem0, sem1)
    lanes = lax.iota(jnp.int32, _L)

    copies = [None, None]
    copies[0] = pltpu.async_copy(x_hbm.at[row0], buf0, sem0)
    resvec = jnp.zeros((_L,), jnp.int32)
    for r in range(_RPW):
        cur = r % 2
        nxt = (r + 1) % 2
        if r + 1 < _RPW:
            copies[nxt] = pltpu.async_copy(
                x_hbm.at[row0 + r + 1], bufs[nxt], sems[nxt])
        copies[cur].wait()
        a = _row_argmax(bufs[cur])
        resvec = jnp.where(lanes == (sid & 3) * _RPW + r, a, resvec)
    res[...] = resvec
    pltpu.sync_copy(res, shared.at[pl.ds(pl.multiple_of(sid * _L, _L), _L)])
    plsc.subcore_barrier()

    grp = sid & 12
    pltpu.sync_copy(
        shared.at[pl.ds(pl.multiple_of(grp * _L, 4 * _L), 4 * _L)], gbuf)
    hi2 = lanes >> 2
    combined = gbuf[pl.ds(0, _L)]
    for j in range(1, 4):
        rowj = gbuf[pl.ds(j * _L, _L)]
        combined = jnp.where(hi2 == j, rowj, combined)
    outv[...] = combined

    @pl.when((sid & 3) == 0)
    def _():
        off = pl.multiple_of((cid * 16 + sid) * _RPW, _L)
        pltpu.sync_copy(outv, out_hbm.at[pl.ds(off, _L)])


def kernel(tensor):
    return _argmax_sc(tensor)


# speedup vs baseline: 1.7314x; 1.5133x over previous
import functools

import jax
import jax.numpy as jnp
from jax import lax
from jax.experimental import pallas as pl
from jax.experimental.pallas import tpu as pltpu
from jax.experimental.pallas import tpu_sc as plsc

_R = 128
_C = 32768
_L = 16
_NW = 32
_RPW = _R // _NW
_NCHUNK = _C // _L


_NACC = 4
_NITER = _NCHUNK // _NACC


def _row_argmax(buf):
    lanes = lax.iota(jnp.int32, _L)

    def step(t, carry):
        ms, cs = carry
        tvec = lax.broadcast(t, (_L,))
        base = t * (_L * _NACC)
        nms, ncs = [], []
        for k in range(_NACC):
            x = buf[pl.ds(base + k * _L, _L)]
            gt = x > ms[k]
            nms.append(jnp.where(gt, x, ms[k]))
            ncs.append(jnp.where(gt, tvec, cs[k]))
        return tuple(nms), tuple(ncs)

    ms0 = tuple(buf[pl.ds(k * _L, _L)] for k in range(_NACC))
    cs0 = tuple(jnp.zeros((_L,), jnp.int32) for _ in range(_NACC))
    ms, cs = lax.fori_loop(1, _NITER, step, (ms0, cs0), unroll=4)

    bm = ms[0]
    bidx = cs[0] * (_NACC * _L) + lanes
    for k in range(1, _NACC):
        idxk = cs[k] * (_NACC * _L) + (k * _L) + lanes
        better = (ms[k] > bm) | ((ms[k] == bm) & (idxk < bidx))
        bm = jnp.where(better, ms[k], bm)
        bidx = jnp.where(better, idxk, bidx)
    sm = bm[0]
    si = bidx[0]
    for j in range(1, _L):
        mv = bm[j]
        iv = bidx[j]
        better = (mv > sm) | ((mv == sm) & (iv < si))
        sm = jnp.where(better, mv, sm)
        si = jnp.where(better, iv, si)
    return si


@functools.partial(
    pl.kernel,
    mesh=plsc.VectorSubcoreMesh(core_axis_name="c", subcore_axis_name="s"),
    out_type=jax.ShapeDtypeStruct((_R,), jnp.int32),
    scratch_types=[
        pltpu.VMEM((_C,), jnp.float32),
        pltpu.VMEM((_C,), jnp.float32),
        pltpu.VMEM((_L,), jnp.int32),
        pltpu.VMEM((4 * _L,), jnp.int32),
        pltpu.VMEM((_L,), jnp.int32),
        pltpu.VMEM_SHARED((16 * _L,), jnp.int32),
        pltpu.SemaphoreType.DMA,
        pltpu.SemaphoreType.DMA,
    ],
)
def _argmax_sc(x_hbm, out_hbm, buf0, buf1, res, gbuf, outv, shared,
               sem0, sem1):
    cid = lax.axis_index("c")
    sid = lax.axis_index("s")
    wid = cid * 16 + sid
    row0 = wid * _RPW
    bufs = (buf0, buf1)
    sems = (sem0, sem1)
    lanes = lax.iota(jnp.int32, _L)

    copies = [None, None]
    copies[0] = pltpu.async_copy(x_hbm.at[row0], buf0, sem0)
    resvec = jnp.zeros((_L,), jnp.int32)
    for r in range(_RPW):
        cur = r % 2
        nxt = (r + 1) % 2
        if r + 1 < _RPW:
            copies[nxt] = pltpu.async_copy(
                x_hbm.at[row0 + r + 1], bufs[nxt], sems[nxt])
        copies[cur].wait()
        a = _row_argmax(bufs[cur])
        resvec = jnp.where(lanes == (sid & 3) * _RPW + r, a, resvec)
    res[...] = resvec
    pltpu.sync_copy(res, shared.at[pl.ds(pl.multiple_of(sid * _L, _L), _L)])
    plsc.subcore_barrier()

    grp = sid & 12
    pltpu.sync_copy(
        shared.at[pl.ds(pl.multiple_of(grp * _L, 4 * _L), 4 * _L)], gbuf)
    hi2 = lanes >> 2
    combined = gbuf[pl.ds(0, _L)]
    for j in range(1, 4):
        rowj = gbuf[pl.ds(j * _L, _L)]
        combined = jnp.where(hi2 == j, rowj, combined)
    outv[...] = combined

    @pl.when((sid & 3) == 0)
    def _():
        off = pl.multiple_of((cid * 16 + sid) * _RPW, _L)
        pltpu.sync_copy(outv, out_hbm.at[pl.ds(off, _L)])


_TCR = 8
_TCB = 128


def _tc_body(x_ref, o_ref):
    nblk = x_ref.shape[1] // _TCB
    lane = lax.broadcasted_iota(jnp.int32, (_TCR, _TCB), 1)

    def step(j, carry):
        m, c = carry
        x = x_ref[:, pl.ds(j * _TCB, _TCB)]
        gt = x > m
        m = jnp.where(gt, x, m)
        c = jnp.where(gt, j, c)
        return m, c

    m0 = x_ref[:, pl.ds(0, _TCB)]
    c0 = jnp.zeros((_TCR, _TCB), jnp.int32)
    m, c = lax.fori_loop(1, nblk, step, (m0, c0), unroll=8)
    rm = jnp.max(m, axis=1)
    cand = jnp.where(m == rm[:, None], c * _TCB + lane, jnp.int32(2 ** 30))
    o_ref[0, 0, :] = jnp.min(cand, axis=1)


def _argmax_tc(x):
    rows = x.shape[0]
    grid = rows // _TCR
    out = pl.pallas_call(
        _tc_body,
        grid=(grid,),
        in_specs=[pl.BlockSpec((_TCR, _C), lambda i: (i, 0))],
        out_specs=pl.BlockSpec((1, 1, _TCR), lambda i: (i, 0, 0)),
        out_shape=jax.ShapeDtypeStruct((grid, 1, _TCR), jnp.int32),
    )(x)
    return out.reshape(rows)


def kernel(tensor):
    return _argmax_tc(tensor)


# speedup vs baseline: 1.8389x; 1.0621x over previous
import functools

import jax
import jax.numpy as jnp
from jax import lax
from jax.experimental import pallas as pl
from jax.experimental.pallas import tpu as pltpu
from jax.experimental.pallas import tpu_sc as plsc

_R = 128
_C = 32768
_L = 16
_NW = 32
_RPW = _R // _NW
_NCHUNK = _C // _L


_NACC = 4
_NITER = _NCHUNK // _NACC


def _row_argmax(buf):
    lanes = lax.iota(jnp.int32, _L)

    def step(t, carry):
        ms, cs = carry
        tvec = lax.broadcast(t, (_L,))
        base = t * (_L * _NACC)
        nms, ncs = [], []
        for k in range(_NACC):
            x = buf[pl.ds(base + k * _L, _L)]
            gt = x > ms[k]
            nms.append(jnp.where(gt, x, ms[k]))
            ncs.append(jnp.where(gt, tvec, cs[k]))
        return tuple(nms), tuple(ncs)

    ms0 = tuple(buf[pl.ds(k * _L, _L)] for k in range(_NACC))
    cs0 = tuple(jnp.zeros((_L,), jnp.int32) for _ in range(_NACC))
    ms, cs = lax.fori_loop(1, _NITER, step, (ms0, cs0), unroll=4)

    bm = ms[0]
    bidx = cs[0] * (_NACC * _L) + lanes
    for k in range(1, _NACC):
        idxk = cs[k] * (_NACC * _L) + (k * _L) + lanes
        better = (ms[k] > bm) | ((ms[k] == bm) & (idxk < bidx))
        bm = jnp.where(better, ms[k], bm)
        bidx = jnp.where(better, idxk, bidx)
    sm = bm[0]
    si = bidx[0]
    for j in range(1, _L):
        mv = bm[j]
        iv = bidx[j]
        better = (mv > sm) | ((mv == sm) & (iv < si))
        sm = jnp.where(better, mv, sm)
        si = jnp.where(better, iv, si)
    return si


@functools.partial(
    pl.kernel,
    mesh=plsc.VectorSubcoreMesh(core_axis_name="c", subcore_axis_name="s"),
    out_type=jax.ShapeDtypeStruct((_R,), jnp.int32),
    scratch_types=[
        pltpu.VMEM((_C,), jnp.float32),
        pltpu.VMEM((_C,), jnp.float32),
        pltpu.VMEM((_L,), jnp.int32),
        pltpu.VMEM((4 * _L,), jnp.int32),
        pltpu.VMEM((_L,), jnp.int32),
        pltpu.VMEM_SHARED((16 * _L,), jnp.int32),
        pltpu.SemaphoreType.DMA,
        pltpu.SemaphoreType.DMA,
    ],
)
def _argmax_sc(x_hbm, out_hbm, buf0, buf1, res, gbuf, outv, shared,
               sem0, sem1):
    cid = lax.axis_index("c")
    sid = lax.axis_index("s")
    wid = cid * 16 + sid
    row0 = wid * _RPW
    bufs = (buf0, buf1)
    sems = (sem0, sem1)
    lanes = lax.iota(jnp.int32, _L)

    copies = [None, None]
    copies[0] = pltpu.async_copy(x_hbm.at[row0], buf0, sem0)
    resvec = jnp.zeros((_L,), jnp.int32)
    for r in range(_RPW):
        cur = r % 2
        nxt = (r + 1) % 2
        if r + 1 < _RPW:
            copies[nxt] = pltpu.async_copy(
                x_hbm.at[row0 + r + 1], bufs[nxt], sems[nxt])
        copies[cur].wait()
        a = _row_argmax(bufs[cur])
        resvec = jnp.where(lanes == (sid & 3) * _RPW + r, a, resvec)
    res[...] = resvec
    pltpu.sync_copy(res, shared.at[pl.ds(pl.multiple_of(sid * _L, _L), _L)])
    plsc.subcore_barrier()

    grp = sid & 12
    pltpu.sync_copy(
        shared.at[pl.ds(pl.multiple_of(grp * _L, 4 * _L), 4 * _L)], gbuf)
    hi2 = lanes >> 2
    combined = gbuf[pl.ds(0, _L)]
    for j in range(1, 4):
        rowj = gbuf[pl.ds(j * _L, _L)]
        combined = jnp.where(hi2 == j, rowj, combined)
    outv[...] = combined

    @pl.when((sid & 3) == 0)
    def _():
        off = pl.multiple_of((cid * 16 + sid) * _RPW, _L)
        pltpu.sync_copy(outv, out_hbm.at[pl.ds(off, _L)])


_TCR = 8
_TCB = 128


_TCACC = 4


def _tc_body(x_ref, o_ref):
    niter = x_ref.shape[1] // (_TCB * _TCACC)
    lane = lax.broadcasted_iota(jnp.int32, (_TCR, _TCB), 1)

    def step(t, carry):
        ms, cs = carry
        base = t * (_TCB * _TCACC)
        nms, ncs = [], []
        for k in range(_TCACC):
            x = x_ref[:, pl.ds(base + k * _TCB, _TCB)]
            gt = x > ms[k]
            nms.append(jnp.where(gt, x, ms[k]))
            ncs.append(jnp.where(gt, t, cs[k]))
        return tuple(nms), tuple(ncs)

    ms0 = tuple(x_ref[:, pl.ds(k * _TCB, _TCB)] for k in range(_TCACC))
    cs0 = tuple(jnp.zeros((_TCR, _TCB), jnp.int32) for _ in range(_TCACC))
    ms, cs = lax.fori_loop(1, niter, step, (ms0, cs0), unroll=4)

    stride = _TCACC * _TCB
    bm = ms[0]
    bidx = cs[0] * stride + lane
    for k in range(1, _TCACC):
        idxk = cs[k] * stride + (k * _TCB) + lane
        better = (ms[k] > bm) | ((ms[k] == bm) & (idxk < bidx))
        bm = jnp.where(better, ms[k], bm)
        bidx = jnp.where(better, idxk, bidx)
    rm = jnp.max(bm, axis=1)
    cand = jnp.where(bm == rm[:, None], bidx, jnp.int32(2 ** 30))
    o_ref[0, 0, :] = jnp.min(cand, axis=1)


def _argmax_tc(x):
    rows = x.shape[0]
    grid = rows // _TCR
    out = pl.pallas_call(
        _tc_body,
        grid=(grid,),
        in_specs=[pl.BlockSpec((_TCR, _C), lambda i: (i, 0))],
        out_specs=pl.BlockSpec((1, 1, _TCR), lambda i: (i, 0, 0)),
        out_shape=jax.ShapeDtypeStruct((grid, 1, _TCR), jnp.int32),
    )(x)
    return out.reshape(rows)


def kernel(tensor):
    return _argmax_tc(tensor)


# speedup vs baseline: 2.0524x; 1.1161x over previous
import functools

import jax
import jax.numpy as jnp
from jax import lax
from jax.experimental import pallas as pl
from jax.experimental.pallas import tpu as pltpu
from jax.experimental.pallas import tpu_sc as plsc

_R = 128
_C = 32768
_L = 16
_NW = 32
_RPW = _R // _NW
_NCHUNK = _C // _L


_NACC = 4
_NITER = _NCHUNK // _NACC


def _row_argmax(buf):
    lanes = lax.iota(jnp.int32, _L)

    def step(t, carry):
        ms, cs = carry
        tvec = lax.broadcast(t, (_L,))
        base = t * (_L * _NACC)
        nms, ncs = [], []
        for k in range(_NACC):
            x = buf[pl.ds(base + k * _L, _L)]
            gt = x > ms[k]
            nms.append(jnp.where(gt, x, ms[k]))
            ncs.append(jnp.where(gt, tvec, cs[k]))
        return tuple(nms), tuple(ncs)

    ms0 = tuple(buf[pl.ds(k * _L, _L)] for k in range(_NACC))
    cs0 = tuple(jnp.zeros((_L,), jnp.int32) for _ in range(_NACC))
    ms, cs = lax.fori_loop(1, _NITER, step, (ms0, cs0), unroll=4)

    bm = ms[0]
    bidx = cs[0] * (_NACC * _L) + lanes
    for k in range(1, _NACC):
        idxk = cs[k] * (_NACC * _L) + (k * _L) + lanes
        better = (ms[k] > bm) | ((ms[k] == bm) & (idxk < bidx))
        bm = jnp.where(better, ms[k], bm)
        bidx = jnp.where(better, idxk, bidx)
    sm = bm[0]
    si = bidx[0]
    for j in range(1, _L):
        mv = bm[j]
        iv = bidx[j]
        better = (mv > sm) | ((mv == sm) & (iv < si))
        sm = jnp.where(better, mv, sm)
        si = jnp.where(better, iv, si)
    return si


@functools.partial(
    pl.kernel,
    mesh=plsc.VectorSubcoreMesh(core_axis_name="c", subcore_axis_name="s"),
    out_type=jax.ShapeDtypeStruct((_R,), jnp.int32),
    scratch_types=[
        pltpu.VMEM((_C,), jnp.float32),
        pltpu.VMEM((_C,), jnp.float32),
        pltpu.VMEM((_L,), jnp.int32),
        pltpu.VMEM((4 * _L,), jnp.int32),
        pltpu.VMEM((_L,), jnp.int32),
        pltpu.VMEM_SHARED((16 * _L,), jnp.int32),
        pltpu.SemaphoreType.DMA,
        pltpu.SemaphoreType.DMA,
    ],
)
def _argmax_sc(x_hbm, out_hbm, buf0, buf1, res, gbuf, outv, shared,
               sem0, sem1):
    cid = lax.axis_index("c")
    sid = lax.axis_index("s")
    wid = cid * 16 + sid
    row0 = wid * _RPW
    bufs = (buf0, buf1)
    sems = (sem0, sem1)
    lanes = lax.iota(jnp.int32, _L)

    copies = [None, None]
    copies[0] = pltpu.async_copy(x_hbm.at[row0], buf0, sem0)
    resvec = jnp.zeros((_L,), jnp.int32)
    for r in range(_RPW):
        cur = r % 2
        nxt = (r + 1) % 2
        if r + 1 < _RPW:
            copies[nxt] = pltpu.async_copy(
                x_hbm.at[row0 + r + 1], bufs[nxt], sems[nxt])
        copies[cur].wait()
        a = _row_argmax(bufs[cur])
        resvec = jnp.where(lanes == (sid & 3) * _RPW + r, a, resvec)
    res[...] = resvec
    pltpu.sync_copy(res, shared.at[pl.ds(pl.multiple_of(sid * _L, _L), _L)])
    plsc.subcore_barrier()

    grp = sid & 12
    pltpu.sync_copy(
        shared.at[pl.ds(pl.multiple_of(grp * _L, 4 * _L), 4 * _L)], gbuf)
    hi2 = lanes >> 2
    combined = gbuf[pl.ds(0, _L)]
    for j in range(1, 4):
        rowj = gbuf[pl.ds(j * _L, _L)]
        combined = jnp.where(hi2 == j, rowj, combined)
    outv[...] = combined

    @pl.when((sid & 3) == 0)
    def _():
        off = pl.multiple_of((cid * 16 + sid) * _RPW, _L)
        pltpu.sync_copy(outv, out_hbm.at[pl.ds(off, _L)])


_TCR = 8
_TCB = 128


_TCACC = 4


def _tc_body(x_ref, o_ref):
    niter = x_ref.shape[1] // (_TCB * _TCACC)
    lane = lax.broadcasted_iota(jnp.int32, (_TCR, _TCB), 1)

    ms = [x_ref[:, pl.ds(k * _TCB, _TCB)] for k in range(_TCACC)]
    cs = [jnp.zeros((_TCR, _TCB), jnp.int32) for _ in range(_TCACC)]
    for t in range(1, niter):
        base = t * (_TCB * _TCACC)
        for k in range(_TCACC):
            x = x_ref[:, pl.ds(base + k * _TCB, _TCB)]
            gt = x > ms[k]
            ms[k] = jnp.where(gt, x, ms[k])
            cs[k] = jnp.where(gt, t, cs[k])

    stride = _TCACC * _TCB
    bm = ms[0]
    bidx = cs[0] * stride + lane
    for k in range(1, _TCACC):
        idxk = cs[k] * stride + (k * _TCB) + lane
        better = (ms[k] > bm) | ((ms[k] == bm) & (idxk < bidx))
        bm = jnp.where(better, ms[k], bm)
        bidx = jnp.where(better, idxk, bidx)
    rm = jnp.max(bm, axis=1)
    cand = jnp.where(bm == rm[:, None], bidx, jnp.int32(2 ** 30))
    o_ref[0, 0, :] = jnp.min(cand, axis=1)


def _argmax_tc(x):
    rows = x.shape[0]
    grid = rows // _TCR
    out = pl.pallas_call(
        _tc_body,
        grid=(grid,),
        in_specs=[pl.BlockSpec((_TCR, _C), lambda i: (i, 0))],
        out_specs=pl.BlockSpec((1, 1, _TCR), lambda i: (i, 0, 0)),
        out_shape=jax.ShapeDtypeStruct((grid, 1, _TCR), jnp.int32),
    )(x)
    return out.reshape(rows)


def kernel(tensor):
    return _argmax_tc(tensor)
